# Initial kernel scaffold; baseline (speedup 1.0000x reference)
import jax
import jax.numpy as jnp
from jax.experimental import pallas as pl


def kernel(x, table):
    raise NotImplementedError("write your pallas kernel here")


# speedup vs baseline: 2.9761x; 2.9761x over previous
import functools

import jax
import jax.numpy as jnp
from jax import lax
from jax.experimental import pallas as pl
from jax.experimental.pallas import tpu as pltpu
from jax.experimental.pallas import tpu_sc as plsc

EMB_DIM = 128
B_TOTAL = 4096 * 50
NUM_WORKERS = 32
CHUNK = 128
ROWS_TOTAL = B_TOTAL // CHUNK
ROWS_PER_W = ROWS_TOTAL // NUM_WORKERS

_mesh = plsc.VectorSubcoreMesh(core_axis_name="c", subcore_axis_name="s")


@functools.partial(
    pl.kernel,
    mesh=_mesh,
    out_type=jax.ShapeDtypeStruct((B_TOTAL, EMB_DIM), jnp.float32),
    scratch_types=[
        pltpu.VMEM((ROWS_PER_W, CHUNK), jnp.int32),
        pltpu.VMEM((CHUNK, EMB_DIM), jnp.float32),
        pltpu.SemaphoreType.DMA,
    ],
)
def _emb_lookup(idx_hbm, table_hbm, out_hbm, idx_v, rows_v, sem):
    wid = lax.axis_index("s") * 2 + lax.axis_index("c")
    row0 = wid * ROWS_PER_W
    pltpu.sync_copy(idx_hbm.at[wid], idx_v)

    def body(j, carry):
        pltpu.async_copy(table_hbm.at[idx_v.at[j]], rows_v, sem).wait()
        pltpu.sync_copy(rows_v, out_hbm.at[pl.ds((row0 + j) * CHUNK, CHUNK)])
        return carry

    lax.fori_loop(0, ROWS_PER_W, body, 0)


@jax.jit
def kernel(x, table):
    idx = x.reshape(NUM_WORKERS, ROWS_PER_W, CHUNK).astype(jnp.int32)
    out = _emb_lookup(idx, table)
    return out.reshape(x.shape[0], x.shape[1], EMB_DIM)


# speedup vs baseline: 3.3710x; 1.1327x over previous
import functools

import jax
import jax.numpy as jnp
from jax import lax
from jax.experimental import pallas as pl
from jax.experimental.pallas import tpu as pltpu
from jax.experimental.pallas import tpu_sc as plsc

EMB_DIM = 128
B_TOTAL = 4096 * 50
NUM_WORKERS = 32
CHUNK = 128
ROWS_TOTAL = B_TOTAL // CHUNK
ROWS_PER_W = ROWS_TOTAL // NUM_WORKERS
NBUF = 6
GD = 3

_mesh = plsc.VectorSubcoreMesh(core_axis_name="c", subcore_axis_name="s")


@functools.partial(
    pl.kernel,
    mesh=_mesh,
    out_type=jax.ShapeDtypeStruct((B_TOTAL, EMB_DIM), jnp.float32),
    scratch_types=[
        pltpu.VMEM((ROWS_PER_W, CHUNK), jnp.int32),
        pltpu.VMEM((NBUF, CHUNK, EMB_DIM), jnp.float32),
        pltpu.SemaphoreType.DMA,
        pltpu.SemaphoreType.DMA,
    ],
)
def _emb_lookup(idx_hbm, table_hbm, out_hbm, idx_v, rows_v, gsem, osem):
    wid = lax.axis_index("s") * 2 + lax.axis_index("c")
    row0 = wid * ROWS_PER_W
    pltpu.sync_copy(idx_hbm.at[wid], idx_v)

    def gfire(s, b):
        return pltpu.async_copy(table_hbm.at[idx_v.at[s]], rows_v.at[b], gsem)

    def ofire(s, b):
        return pltpu.async_copy(
            rows_v.at[b], out_hbm.at[pl.ds((row0 + s) * CHUNK, CHUNK)], osem)

    def gwait():
        pltpu.make_async_copy(
            table_hbm.at[idx_v.at[0]], rows_v.at[0], gsem).wait()

    def owait():
        pltpu.make_async_copy(
            rows_v.at[0], out_hbm.at[pl.ds(row0 * CHUNK, CHUNK)], osem).wait()

    for b in range(GD):
        gfire(b, b)
    for s in range(NBUF - GD):
        gfire(s + GD, s + GD)
        gwait()
        ofire(s, s)

    def body(s, carry):
        owait()
        gfire(s + GD, lax.rem(s + GD, NBUF))
        gwait()
        ofire(s, lax.rem(s, NBUF))
        return carry

    lax.fori_loop(NBUF - GD, ROWS_PER_W - GD, body, 0)

    for k in range(GD):
        s = ROWS_PER_W - GD + k
        gwait()
        ofire(s, lax.rem(s, NBUF))
    for _ in range(NBUF):
        owait()


@jax.jit
def kernel(x, table):
    idx = x.reshape(NUM_WORKERS, ROWS_PER_W, CHUNK).astype(jnp.int32)
    out = _emb_lookup(idx, table)
    return out.reshape(x.shape[0], x.shape[1], EMB_DIM)


# speedup vs baseline: 6.0070x; 1.7820x over previous
import functools

import jax
import jax.numpy as jnp
from jax import lax
from jax.experimental import pallas as pl
from jax.experimental.pallas import tpu as pltpu
from jax.experimental.pallas import tpu_sc as plsc

EMB_DIM = 128
BATCH = 4096
SEQ = 50
NUM_WORKERS = 32
BPW = BATCH // NUM_WORKERS
NBUF = 8
GD = 4

_mesh = plsc.VectorSubcoreMesh(core_axis_name="c", subcore_axis_name="s")


@functools.partial(
    pl.kernel,
    mesh=_mesh,
    out_type=jax.ShapeDtypeStruct((BATCH, SEQ, EMB_DIM), jnp.float32),
    scratch_types=[
        pltpu.VMEM((BPW, SEQ), jnp.int32),
        pltpu.VMEM((NBUF, SEQ, EMB_DIM), jnp.float32),
        pltpu.SemaphoreType.DMA,
        pltpu.SemaphoreType.DMA,
    ],
)
def _emb_lookup(idx_hbm, table_hbm, out_hbm, idx_v, rows_v, gsem, osem):
    wid = lax.axis_index("s") * 2 + lax.axis_index("c")
    bat0 = wid * BPW
    pltpu.sync_copy(idx_hbm.at[pl.ds(bat0, BPW)], idx_v)

    def gfire(s, b):
        return pltpu.async_copy(table_hbm.at[idx_v.at[s]], rows_v.at[b], gsem)

    def ofire(s, b):
        return pltpu.async_copy(rows_v.at[b], out_hbm.at[bat0 + s], osem)

    def gwait():
        pltpu.make_async_copy(
            table_hbm.at[idx_v.at[0]], rows_v.at[0], gsem).wait()

    def owait():
        pltpu.make_async_copy(rows_v.at[0], out_hbm.at[bat0], osem).wait()

    for b in range(GD):
        gfire(b, b)
    for s in range(NBUF - GD):
        gfire(s + GD, s + GD)
        gwait()
        ofire(s, s)

    def body(s, carry):
        owait()
        gfire(s + GD, lax.rem(s + GD, NBUF))
        gwait()
        ofire(s, lax.rem(s, NBUF))
        return carry

    lax.fori_loop(NBUF - GD, BPW - GD, body, 0)

    for k in range(GD):
        s = BPW - GD + k
        gwait()
        ofire(s, lax.rem(s, NBUF))
    for _ in range(NBUF):
        owait()


@jax.jit
def kernel(x, table):
    return _emb_lookup(x.astype(jnp.int32), table)
